# baseline (speedup 1.0000x reference)
import functools

import jax
import jax.numpy as jnp
from jax import lax
from jax.experimental import pallas as pl
from jax.experimental.pallas import tpu as pltpu
from jax.experimental.pallas import tpu_sc as plsc

T = 1024
B = 32
NSLOT = 3
F = 512
DV = 128
R = B * NSLOT
NCORE, NSUB = 2, 16
NW = NCORE * NSUB
T_PER_W = T // NW
LANES = 16


def _prep_body(x_ref, vel_ref, ctrl_ref, cidx_ref, table2_ref, pe2_ref):
    slot = lax.broadcasted_iota(jnp.int32, (1, R), 1) % NSLOT
    cidx_ref[...] = x_ref[...] + jnp.where(slot == 0, 0, DV)

    vel = vel_ref[...]
    ctrl = ctrl_ref[...]
    vn = jnp.max(jnp.abs(vel), axis=1, keepdims=True)
    cn = jnp.max(jnp.abs(ctrl), axis=1, keepdims=True)
    vscale = jnp.where(vn > 1.0, 1.0 / vn, 1.0)
    cscale = jnp.where(cn > 127.0, 127.0 / cn, 1.0)
    table2_ref[0:DV, :] = vel * (2.0 * vscale)
    table2_ref[DV:2 * DV, :] = ctrl * (2.0 * cscale)

    pos = lax.broadcasted_iota(jnp.int32, (T, F), 0).astype(jnp.float32)
    col = lax.broadcasted_iota(jnp.int32, (T, F), 1)
    colmod = col % 2
    ang = pos * jnp.exp((col - colmod).astype(jnp.float32) * (-4.0 / F))
    pe2_ref[...] = 2.0 * jnp.where(colmod == 0, jnp.sin(ang), jnp.cos(ang))


def _prep(x2, vel_table, ctrl_table):
    return pl.pallas_call(
        _prep_body,
        out_shape=[
            jax.ShapeDtypeStruct((T, R), jnp.int32),
            jax.ShapeDtypeStruct((2 * DV, F), jnp.float32),
            jax.ShapeDtypeStruct((T, F), jnp.float32),
        ],
    )(x2, vel_table, ctrl_table)


B_CH = 8
CH_PER_T = B // B_CH
R_CH = B_CH * NSLOT
NU = T_PER_W * CH_PER_T
NBUF = 2


def _sc_body(cidx_hbm, table2_hbm, pe2_hbm, out_hbm, cidx_v, pe_v,
             rows0, rows1, obuf0, obuf1, gsem0, gsem1, wsem0, wsem1):
    cid = lax.axis_index("c")
    sid = lax.axis_index("s")
    wid = cid * NSUB + sid
    t0 = wid * T_PER_W
    rows = (rows0, rows1)
    obuf = (obuf0, obuf1)
    gsem = (gsem0, gsem1)
    wsem = (wsem0, wsem1)

    pltpu.sync_copy(cidx_hbm.at[pl.ds(t0 * R, T_PER_W * R)], cidx_v)
    pltpu.sync_copy(pe2_hbm.at[pl.ds(t0, T_PER_W)], pe_v)

    def gather(u, k):
        idx = cidx_v.at[pl.ds(u * R_CH, R_CH)]
        return pltpu.make_async_copy(table2_hbm.at[idx], rows[k], gsem[k])

    def write(u, k):
        i = u // CH_PER_T
        h = u % CH_PER_T
        dst = out_hbm.at[t0 + i, pl.ds(h * B_CH, B_CH)]
        return pltpu.make_async_copy(obuf[k], dst, wsem[k])

    gather(0, 0).start()
    gather(1, 1).start()

    def pair(p, carry):
        u0 = p * NBUF
        for k in range(NBUF):
            u = u0 + k
            i = u // CH_PER_T
            gather(u, k).wait()

            @pl.when(p > 0)
            def _():
                write(u - NBUF, k).wait()

            pev = [pe_v[i, pl.ds(v * LANES, LANES)] for v in range(F // LANES)]

            @plsc.parallel_loop(0, B_CH, unroll=2)
            def row(b):
                for s in range(NSLOT):
                    for v in range(F // LANES):
                        sl = pl.ds(v * LANES, LANES)
                        obuf[k][b, s, sl] = (rows[k][b * NSLOT + s, sl]
                                             + pev[v])

            @pl.when(u + NBUF < NU)
            def _():
                gather(u + NBUF, k).start()

            write(u, k).start()
        return carry

    lax.fori_loop(0, NU // NBUF, pair, 0)
    write(NU - 2, 0).wait()
    write(NU - 1, 1).wait()


@functools.cache
def _sc_embed():
    return pl.kernel(
        _sc_body,
        out_type=jax.ShapeDtypeStruct((T, B, NSLOT, F), jnp.float32),
        mesh=plsc.VectorSubcoreMesh(core_axis_name="c", subcore_axis_name="s",
                                    num_cores=NCORE, num_subcores=NSUB),
        scratch_types=[
            pltpu.VMEM((T_PER_W * R,), jnp.int32),
            pltpu.VMEM((T_PER_W, F), jnp.float32),
            pltpu.VMEM((R_CH, F), jnp.float32),
            pltpu.VMEM((R_CH, F), jnp.float32),
            pltpu.VMEM((B_CH, NSLOT, F), jnp.float32),
            pltpu.VMEM((B_CH, NSLOT, F), jnp.float32),
            pltpu.SemaphoreType.DMA,
            pltpu.SemaphoreType.DMA,
            pltpu.SemaphoreType.DMA,
            pltpu.SemaphoreType.DMA,
        ],
    )


def kernel(x, vel_table, ctrl_table):
    x2 = x.reshape(T, R).astype(jnp.int32)
    cidx, table2, pe2 = _prep(x2, vel_table, ctrl_table)
    return _sc_embed()(cidx.reshape(T * R), table2, pe2)


# speedup vs baseline: 1.0193x; 1.0193x over previous
import functools

import jax
import jax.numpy as jnp
from jax import lax
from jax.experimental import pallas as pl
from jax.experimental.pallas import tpu as pltpu
from jax.experimental.pallas import tpu_sc as plsc

T = 1024
B = 32
NSLOT = 3
F = 512
DV = 128
R = B * NSLOT
NCORE, NSUB = 2, 16
NW = NCORE * NSUB
T_PER_W = T // NW
LANES = 16


def _prep_body(x_ref, vel_ref, ctrl_ref, cidx_ref, table2_ref, pe2_ref):
    slot = lax.broadcasted_iota(jnp.int32, (1, R), 1) % NSLOT
    cidx_ref[...] = x_ref[...] + jnp.where(slot == 0, 0, DV)

    vel = vel_ref[...]
    ctrl = ctrl_ref[...]
    vn = jnp.max(jnp.abs(vel), axis=1, keepdims=True)
    cn = jnp.max(jnp.abs(ctrl), axis=1, keepdims=True)
    vscale = jnp.where(vn > 1.0, 1.0 / vn, 1.0)
    cscale = jnp.where(cn > 127.0, 127.0 / cn, 1.0)
    table2_ref[0:DV, :] = vel * (2.0 * vscale)
    table2_ref[DV:2 * DV, :] = ctrl * (2.0 * cscale)

    pos = lax.broadcasted_iota(jnp.int32, (T, F), 0).astype(jnp.float32)
    col = lax.broadcasted_iota(jnp.int32, (T, F), 1)
    colmod = col % 2
    ang = pos * jnp.exp((col - colmod).astype(jnp.float32) * (-4.0 / F))
    pe2_ref[...] = 2.0 * jnp.where(colmod == 0, jnp.sin(ang), jnp.cos(ang))


def _prep(x2, vel_table, ctrl_table):
    return pl.pallas_call(
        _prep_body,
        out_shape=[
            jax.ShapeDtypeStruct((T, R), jnp.int32),
            jax.ShapeDtypeStruct((2 * DV, F), jnp.float32),
            jax.ShapeDtypeStruct((T, F), jnp.float32),
        ],
    )(x2, vel_table, ctrl_table)


B_CH = 8
CH_PER_T = B // B_CH
R_CH = B_CH * NSLOT
NU = T_PER_W * CH_PER_T
NBUF = 2


def _sc_body(cidx_hbm, table2_hbm, pe2_hbm, out_hbm, cidx_v, pe_v,
             rows0, rows1, obuf0, obuf1, gsem0, gsem1, wsem0, wsem1):
    cid = lax.axis_index("c")
    sid = lax.axis_index("s")
    wid = cid * NSUB + sid
    t0 = wid * T_PER_W
    rows = (rows0, rows1)
    obuf = (obuf0, obuf1)
    gsem = (gsem0, gsem1)
    wsem = (wsem0, wsem1)

    pltpu.sync_copy(cidx_hbm.at[pl.ds(t0 * R, T_PER_W * R)], cidx_v)
    pltpu.sync_copy(pe2_hbm.at[pl.ds(t0, T_PER_W)], pe_v)

    def gather(u, k):
        idx = cidx_v.at[pl.ds(u * R_CH, R_CH)]
        return pltpu.make_async_copy(table2_hbm.at[idx], rows[k], gsem[k])

    def write(u, k):
        i = u // CH_PER_T
        h = u % CH_PER_T
        dst = out_hbm.at[t0 + i, pl.ds(h * B_CH, B_CH)]
        return pltpu.make_async_copy(obuf[k], dst, wsem[k])

    gather(0, 0).start()
    gather(1, 1).start()

    def pair(p, carry):
        u0 = p * NBUF
        for k in range(NBUF):
            u = u0 + k
            i = u // CH_PER_T
            gather(u, k).wait()

            @pl.when(p > 0)
            def _():
                write(u - NBUF, k).wait()

            pev = [pe_v[i, pl.ds(v * LANES, LANES)] for v in range(F // LANES)]

            @plsc.parallel_loop(0, B_CH)
            def row(b):
                for s in range(NSLOT):
                    for v in range(F // LANES):
                        sl = pl.ds(v * LANES, LANES)
                        obuf[k][b, s, sl] = (rows[k][b * NSLOT + s, sl]
                                             + pev[v])

            @pl.when(u + NBUF < NU)
            def _():
                gather(u + NBUF, k).start()

            write(u, k).start()
        return carry

    lax.fori_loop(0, NU // NBUF, pair, 0)
    write(NU - 2, 0).wait()
    write(NU - 1, 1).wait()


@functools.cache
def _sc_embed():
    return pl.kernel(
        _sc_body,
        out_type=jax.ShapeDtypeStruct((T, B, NSLOT, F), jnp.float32),
        mesh=plsc.VectorSubcoreMesh(core_axis_name="c", subcore_axis_name="s",
                                    num_cores=NCORE, num_subcores=NSUB),
        scratch_types=[
            pltpu.VMEM((T_PER_W * R,), jnp.int32),
            pltpu.VMEM((T_PER_W, F), jnp.float32),
            pltpu.VMEM((R_CH, F), jnp.float32),
            pltpu.VMEM((R_CH, F), jnp.float32),
            pltpu.VMEM((B_CH, NSLOT, F), jnp.float32),
            pltpu.VMEM((B_CH, NSLOT, F), jnp.float32),
            pltpu.SemaphoreType.DMA,
            pltpu.SemaphoreType.DMA,
            pltpu.SemaphoreType.DMA,
            pltpu.SemaphoreType.DMA,
        ],
    )


def kernel(x, vel_table, ctrl_table):
    x2 = x.reshape(T, R).astype(jnp.int32)
    cidx, table2, pe2 = _prep(x2, vel_table, ctrl_table)
    return _sc_embed()(cidx.reshape(T * R), table2, pe2)
